# baseline (speedup 1.0000x reference)
import functools

import jax
import jax.numpy as jnp
from jax import lax
from jax.experimental import pallas as pl
from jax.experimental.pallas import tpu as pltpu
from jax.experimental.pallas import tpu_sc as plsc

_LANES = 16
_CHUNK = 128


@functools.lru_cache(maxsize=None)
def _make_sc_gather(batch: int, res: int):
    info = plsc.get_sparse_core_info()
    num_workers = info.num_cores * info.num_subcores
    b_per_w = batch // num_workers
    assert batch % num_workers == 0 and b_per_w % _CHUNK == 0
    n_chunks = b_per_w // _CHUNK
    mesh = plsc.VectorSubcoreMesh(core_axis_name="c", subcore_axis_name="s")

    @functools.partial(
        pl.kernel,
        out_type=jax.ShapeDtypeStruct((batch,), jnp.float32),
        mesh=mesh,
        scratch_types=[
            pltpu.VMEM((b_per_w,), jnp.float32),
            pltpu.VMEM((b_per_w,), jnp.float32),
            pltpu.VMEM((n_chunks, _CHUNK), jnp.int32),
            pltpu.VMEM((n_chunks, _CHUNK), jnp.float32),
            pltpu.SemaphoreType.DMA,
        ],
    )
    def gather_kernel(x0_hbm, x1_hbm, table_hbm, out_hbm,
                      x0_v, x1_v, idx_v, val_v, sem):
        wid = lax.axis_index("s") * info.num_cores + lax.axis_index("c")
        base = wid * b_per_w
        pltpu.sync_copy(x0_hbm.at[pl.ds(base, b_per_w)], x0_v)
        pltpu.sync_copy(x1_hbm.at[pl.ds(base, b_per_w)], x1_v)
        scale = jnp.float32(res - 1)
        per_row = _CHUNK // _LANES
        for i in range(b_per_w // _LANES):
            v0 = x0_v[pl.ds(i * _LANES, _LANES)]
            v1 = x1_v[pl.ds(i * _LANES, _LANES)]
            px = (v0 * scale).astype(jnp.int32)
            py = (v1 * scale).astype(jnp.int32)
            idx_v[i // per_row, pl.ds((i % per_row) * _LANES, _LANES)] = (
                px * res + py)
        copies = [
            pltpu.async_copy(table_hbm.at[idx_v.at[j]], val_v.at[j], sem)
            for j in range(n_chunks)
        ]
        for c in copies:
            c.wait()
        for j in range(n_chunks):
            pltpu.sync_copy(val_v.at[j],
                            out_hbm.at[pl.ds(base + j * _CHUNK, _CHUNK)])

    return gather_kernel


def kernel(x, grid):
    batch = x.shape[0]
    res = grid.shape[0]
    value = _make_sc_gather(batch, res)(x[:, 0], x[:, 1], grid.reshape(-1))
    action = jax.random.uniform(
        jax.random.key(42), (batch, 4), dtype=jnp.float32) * 2.0 - 1.0
    return (action, value[:, None])


# speedup vs baseline: 3.1688x; 3.1688x over previous
import functools

import jax
import jax.numpy as jnp
from jax import lax
from jax.experimental import pallas as pl
from jax.experimental.pallas import tpu as pltpu
from jax.experimental.pallas import tpu_sc as plsc

_LANES = 16
_CHUNK = 128


@functools.lru_cache(maxsize=None)
def _make_sc_gather(batch: int, res: int):
    info = plsc.get_sparse_core_info()
    num_workers = info.num_cores * info.num_subcores
    b_per_w = batch // num_workers
    assert batch % num_workers == 0 and b_per_w % _CHUNK == 0
    n_chunks = b_per_w // _CHUNK
    mesh = plsc.VectorSubcoreMesh(core_axis_name="c", subcore_axis_name="s")

    @functools.partial(
        pl.kernel,
        out_type=jax.ShapeDtypeStruct((batch,), jnp.float32),
        mesh=mesh,
        scratch_types=[
            pltpu.VMEM((b_per_w,), jnp.float32),
            pltpu.VMEM((b_per_w,), jnp.float32),
            pltpu.VMEM((n_chunks, _CHUNK), jnp.int32),
            pltpu.VMEM((n_chunks, _CHUNK), jnp.float32),
            pltpu.SemaphoreType.DMA,
        ],
    )
    def gather_kernel(x0_hbm, x1_hbm, table_hbm, out_hbm,
                      x0_v, x1_v, idx_v, val_v, sem):
        wid = lax.axis_index("s") * info.num_cores + lax.axis_index("c")
        base = wid * b_per_w
        pltpu.sync_copy(x0_hbm.at[pl.ds(base, b_per_w)], x0_v)
        pltpu.sync_copy(x1_hbm.at[pl.ds(base, b_per_w)], x1_v)
        scale = jnp.float32(res - 1)
        per_row = _CHUNK // _LANES
        col_tiles_shift = (res // 128).bit_length() - 1 + 10
        for i in range(b_per_w // _LANES):
            v0 = x0_v[pl.ds(i * _LANES, _LANES)]
            v1 = x1_v[pl.ds(i * _LANES, _LANES)]
            px = (v0 * scale).astype(jnp.int32)
            py = (v1 * scale).astype(jnp.int32)
            flat = (((px >> 3) << col_tiles_shift) + ((py >> 7) << 10)
                    + ((px & 7) << 7) + (py & 127))
            idx_v[i // per_row, pl.ds((i % per_row) * _LANES, _LANES)] = flat
        copies = [
            pltpu.async_copy(table_hbm.at[idx_v.at[j]], val_v.at[j], sem)
            for j in range(n_chunks)
        ]
        for c in copies:
            c.wait()
        for j in range(n_chunks):
            pltpu.sync_copy(val_v.at[j],
                            out_hbm.at[pl.ds(base + j * _CHUNK, _CHUNK)])

    return gather_kernel


def kernel(x, grid):
    batch = x.shape[0]
    res = grid.shape[0]
    table = grid.reshape(res // 8, 8, res // 128, 128).transpose(
        0, 2, 1, 3).reshape(-1)
    value = _make_sc_gather(batch, res)(x[:, 0], x[:, 1], table)
    action = jax.random.uniform(
        jax.random.key(42), (batch, 4), dtype=jnp.float32) * 2.0 - 1.0
    return (action, value[:, None])


# speedup vs baseline: 3.2306x; 1.0195x over previous
import functools

import jax
import jax.numpy as jnp
from jax import lax
from jax.experimental import pallas as pl
from jax.experimental.pallas import tpu as pltpu
from jax.experimental.pallas import tpu_sc as plsc

_LANES = 16
_CHUNK = 128


@functools.lru_cache(maxsize=None)
def _make_sc_gather(batch: int, res: int):
    info = plsc.get_sparse_core_info()
    num_workers = info.num_cores * info.num_subcores
    b_per_w = batch // num_workers
    assert batch % num_workers == 0 and b_per_w % _CHUNK == 0
    n_chunks = b_per_w // _CHUNK
    mesh = plsc.VectorSubcoreMesh(core_axis_name="c", subcore_axis_name="s")

    @functools.partial(
        pl.kernel,
        out_type=jax.ShapeDtypeStruct((batch,), jnp.float32),
        mesh=mesh,
        scratch_types=[
            pltpu.VMEM((b_per_w,), jnp.float32),
            pltpu.VMEM((b_per_w,), jnp.float32),
            pltpu.VMEM((n_chunks, _CHUNK), jnp.int32),
            pltpu.VMEM((b_per_w,), jnp.float32),
            pltpu.SemaphoreType.DMA,
            pltpu.SemaphoreType.DMA,
        ],
    )
    def gather_kernel(x0_hbm, x1_hbm, table_hbm, out_hbm,
                      x0_v, x1_v, idx_v, val_v, in_sem, sem):
        wid = lax.axis_index("s") * info.num_cores + lax.axis_index("c")
        base = wid * b_per_w
        cp0 = pltpu.async_copy(x0_hbm.at[pl.ds(base, b_per_w)], x0_v, in_sem)
        cp1 = pltpu.async_copy(x1_hbm.at[pl.ds(base, b_per_w)], x1_v, in_sem)
        cp0.wait()
        cp1.wait()
        scale = jnp.float32(res - 1)
        per_row = _CHUNK // _LANES
        col_tiles_shift = (res // 128).bit_length() - 1 + 10
        gathers = []
        for i in range(b_per_w // _LANES):
            v0 = x0_v[pl.ds(i * _LANES, _LANES)]
            v1 = x1_v[pl.ds(i * _LANES, _LANES)]
            px = (v0 * scale).astype(jnp.int32)
            py = (v1 * scale).astype(jnp.int32)
            flat = (((px >> 3) << col_tiles_shift) + ((py >> 7) << 10)
                    + ((px & 7) << 7) + (py & 127))
            j, k = divmod(i, per_row)
            idx_v[j, pl.ds(k * _LANES, _LANES)] = flat
            if k == per_row - 1:
                gathers.append(pltpu.async_copy(
                    table_hbm.at[idx_v.at[j]],
                    val_v.at[pl.ds(j * _CHUNK, _CHUNK)], sem))
        for g in gathers:
            g.wait()
        pltpu.sync_copy(val_v, out_hbm.at[pl.ds(base, b_per_w)])

    return gather_kernel


def kernel(x, grid):
    batch = x.shape[0]
    res = grid.shape[0]
    table = grid.reshape(res // 8, 8, res // 128, 128).transpose(
        0, 2, 1, 3).reshape(-1)
    value = _make_sc_gather(batch, res)(x[:, 0], x[:, 1], table)
    action = jax.random.uniform(
        jax.random.key(42), (batch, 4), dtype=jnp.float32) * 2.0 - 1.0
    return (action, value[:, None])


# speedup vs baseline: 3.3229x; 1.0286x over previous
import functools

import jax
import jax.numpy as jnp
from jax import lax
from jax.experimental import pallas as pl
from jax.experimental.pallas import tpu as pltpu
from jax.experimental.pallas import tpu_sc as plsc

_CHUNK = 128


@functools.lru_cache(maxsize=None)
def _make_sc_gather(batch: int):
    info = plsc.get_sparse_core_info()
    num_workers = info.num_cores * info.num_subcores
    b_per_w = batch // num_workers
    assert batch % num_workers == 0 and b_per_w % _CHUNK == 0
    n_chunks = b_per_w // _CHUNK
    mesh = plsc.VectorSubcoreMesh(core_axis_name="c", subcore_axis_name="s")

    @functools.partial(
        pl.kernel,
        out_type=jax.ShapeDtypeStruct((batch,), jnp.float32),
        mesh=mesh,
        scratch_types=[
            pltpu.VMEM((b_per_w,), jnp.int32),
            pltpu.VMEM((b_per_w,), jnp.float32),
            pltpu.SemaphoreType.DMA,
            pltpu.SemaphoreType.DMA,
        ],
    )
    def gather_kernel(idx_hbm, table_hbm, out_hbm, idx_v, val_v, in_sem, sem):
        wid = lax.axis_index("s") * info.num_cores + lax.axis_index("c")
        base = wid * b_per_w
        pltpu.async_copy(idx_hbm.at[pl.ds(base, b_per_w)], idx_v,
                         in_sem).wait()
        gathers = [
            pltpu.async_copy(
                table_hbm.at[idx_v.at[pl.ds(j * _CHUNK, _CHUNK)]],
                val_v.at[pl.ds(j * _CHUNK, _CHUNK)], sem)
            for j in range(n_chunks)
        ]
        for g in gathers:
            g.wait()
        pltpu.sync_copy(val_v, out_hbm.at[pl.ds(base, b_per_w)])

    return gather_kernel


def kernel(x, grid):
    batch = x.shape[0]
    res = grid.shape[0]
    table = grid.reshape(res // 8, 8, res // 128, 128).transpose(
        0, 2, 1, 3).reshape(-1)
    scale = jnp.float32(res - 1)
    px = jnp.floor(x[:, 0] * scale).astype(jnp.int32)
    py = jnp.floor(x[:, 1] * scale).astype(jnp.int32)
    col_tiles_shift = (res // 128).bit_length() - 1 + 10
    idx = (((px >> 3) << col_tiles_shift) + ((py >> 7) << 10)
           + ((px & 7) << 7) + (py & 127))
    value = _make_sc_gather(batch)(idx, table)
    action = jax.random.uniform(
        jax.random.key(42), (batch, 4), dtype=jnp.float32) * 2.0 - 1.0
    return (action, value[:, None])


# speedup vs baseline: 3.3408x; 1.0054x over previous
import functools

import jax
import jax.numpy as jnp
from jax import lax
from jax.experimental import pallas as pl
from jax.experimental.pallas import tpu as pltpu
from jax.experimental.pallas import tpu_sc as plsc

_CHUNK = 128


@functools.lru_cache(maxsize=None)
def _make_sc_gather(batch: int):
    info = plsc.get_sparse_core_info()
    num_workers = info.num_cores * info.num_subcores
    b_per_w = batch // num_workers
    assert batch % num_workers == 0 and b_per_w % _CHUNK == 0
    n_chunks = b_per_w // _CHUNK
    mesh = plsc.VectorSubcoreMesh(core_axis_name="c", subcore_axis_name="s")

    @functools.partial(
        pl.kernel,
        out_type=jax.ShapeDtypeStruct((batch,), jnp.float32),
        mesh=mesh,
        scratch_types=[
            pltpu.VMEM((b_per_w,), jnp.int32),
            pltpu.VMEM((b_per_w,), jnp.float32),
            pltpu.SemaphoreType.DMA,
            pltpu.SemaphoreType.DMA,
        ],
    )
    def gather_kernel(idx_hbm, table_hbm, out_hbm, idx_v, val_v, in_sem, sem):
        wid = lax.axis_index("s") * info.num_cores + lax.axis_index("c")
        base = wid * b_per_w
        pltpu.async_copy(idx_hbm.at[pl.ds(base, b_per_w)], idx_v,
                         in_sem).wait()
        pltpu.async_copy(table_hbm.at[idx_v], val_v, sem).wait()
        pltpu.sync_copy(val_v, out_hbm.at[pl.ds(base, b_per_w)])

    return gather_kernel


def kernel(x, grid):
    batch = x.shape[0]
    res = grid.shape[0]
    table = grid.reshape(res // 8, 8, res // 128, 128).transpose(
        0, 2, 1, 3).reshape(-1)
    scale = jnp.float32(res - 1)
    px = jnp.floor(x[:, 0] * scale).astype(jnp.int32)
    py = jnp.floor(x[:, 1] * scale).astype(jnp.int32)
    col_tiles_shift = (res // 128).bit_length() - 1 + 10
    idx = (((px >> 3) << col_tiles_shift) + ((py >> 7) << 10)
           + ((px & 7) << 7) + (py & 127))
    value = _make_sc_gather(batch)(idx, table)
    action = jax.random.uniform(
        jax.random.key(42), (batch, 4), dtype=jnp.float32) * 2.0 - 1.0
    return (action, value[:, None])
